# Initial kernel scaffold; baseline (speedup 1.0000x reference)
import jax
import jax.numpy as jnp
from jax.experimental import pallas as pl


def kernel(feats, coords, xyz_t, W, ln_gamma, ln_beta):
    raise NotImplementedError("write your pallas kernel here")


# speedup vs baseline: 43.0972x; 43.0972x over previous
import functools

import jax
import jax.numpy as jnp
from jax.experimental import pallas as pl
from jax.experimental.pallas import tpu as pltpu

QS = 0.01


def _ftu_kernel(coords_ref, xyz_ref, feats_ref, w_ref, g_ref, b_ref, out_ref,
                *, nt, m):
    src = coords_ref[0].astype(jnp.float32) * QS
    q = xyz_ref[0]

    d2 = ((q[:, 0:1] - src[0:1, :]) ** 2
          + (q[:, 1:2] - src[1:2, :]) ** 2
          + (q[:, 2:3] - src[2:3, :]) ** 2)

    iota = jax.lax.broadcasted_iota(jnp.int32, (nt, m), 1)
    inf = jnp.float32(jnp.inf)

    d = d2
    dists = []
    idxs = []
    for _ in range(3):
        mn = jnp.min(d, axis=1, keepdims=True)
        ix = jnp.min(jnp.where(d == mn, iota, m), axis=1, keepdims=True)
        dists.append(mn)
        idxs.append(ix)
        d = jnp.where(iota == ix, inf, d)

    r0 = 1.0 / (dists[0] + 1e-8)
    r1 = 1.0 / (dists[1] + 1e-8)
    r2 = 1.0 / (dists[2] + 1e-8)
    inv_norm = 1.0 / (r0 + r1 + r2)

    zero = jnp.float32(0.0)
    sel = (jnp.where(iota == idxs[0], r0 * inv_norm, zero)
           + jnp.where(iota == idxs[1], r1 * inv_norm, zero)
           + jnp.where(iota == idxs[2], r2 * inv_norm, zero))

    gathered = jax.lax.dot(sel, feats_ref[0],
                           preferred_element_type=jnp.float32)
    interp = jax.lax.dot(gathered, w_ref[...],
                         preferred_element_type=jnp.float32)

    mu = jnp.mean(interp, axis=1, keepdims=True)
    xc = interp - mu
    var = jnp.mean(xc * xc, axis=1, keepdims=True)
    xn = xc / jnp.sqrt(var + 1e-6) * g_ref[0] + b_ref[0]

    inv_sqrt2 = jnp.float32(0.7071067811865476)
    act = 0.5 * xn * (1.0 + jax.lax.erf(xn * inv_sqrt2))

    out_ref[0] = act.T


def kernel(feats, coords, xyz_t, W, ln_gamma, ln_beta):
    B, M, INP = feats.shape
    _, N, _ = xyz_t.shape
    OUT = W.shape[1]
    NT = 256
    assert N % NT == 0

    coords_t = jnp.transpose(coords, (0, 2, 1))
    gamma2 = ln_gamma.reshape(1, OUT)
    beta2 = ln_beta.reshape(1, OUT)

    grid = (B, N // NT)
    out = pl.pallas_call(
        functools.partial(_ftu_kernel, nt=NT, m=M),
        grid=grid,
        in_specs=[
            pl.BlockSpec((1, 3, M), lambda b, n: (b, 0, 0)),
            pl.BlockSpec((1, NT, 3), lambda b, n: (b, n, 0)),
            pl.BlockSpec((1, M, INP), lambda b, n: (b, 0, 0)),
            pl.BlockSpec((INP, OUT), lambda b, n: (0, 0)),
            pl.BlockSpec((1, OUT), lambda b, n: (0, 0)),
            pl.BlockSpec((1, OUT), lambda b, n: (0, 0)),
        ],
        out_specs=pl.BlockSpec((1, OUT, NT), lambda b, n: (b, 0, n)),
        out_shape=jax.ShapeDtypeStruct((B, OUT, N), jnp.float32),
        compiler_params=pltpu.CompilerParams(
            dimension_semantics=("arbitrary", "arbitrary"),
        ),
    )(coords_t, xyz_t, feats, W, gamma2, beta2)
    return out


# speedup vs baseline: 43.1120x; 1.0003x over previous
import functools

import jax
import jax.numpy as jnp
from jax.experimental import pallas as pl
from jax.experimental.pallas import tpu as pltpu

QS = 0.01


def _ftu_kernel(coords_ref, xyz_ref, feats_ref, w_ref, g_ref, b_ref, out_ref,
                *, nt, m):
    src = coords_ref[0].astype(jnp.float32) * QS
    q = xyz_ref[0]

    d2 = ((q[:, 0:1] - src[0:1, :]) ** 2
          + (q[:, 1:2] - src[1:2, :]) ** 2
          + (q[:, 2:3] - src[2:3, :]) ** 2)

    iota = jax.lax.broadcasted_iota(jnp.int32, (nt, m), 1)
    inf = jnp.float32(jnp.inf)

    d = d2
    dists = []
    idxs = []
    for _ in range(3):
        mn = jnp.min(d, axis=1, keepdims=True)
        ix = jnp.min(jnp.where(d == mn, iota, m), axis=1, keepdims=True)
        dists.append(mn)
        idxs.append(ix)
        d = jnp.where(iota == ix, inf, d)

    r0 = 1.0 / (dists[0] + 1e-8)
    r1 = 1.0 / (dists[1] + 1e-8)
    r2 = 1.0 / (dists[2] + 1e-8)
    inv_norm = 1.0 / (r0 + r1 + r2)

    zero = jnp.float32(0.0)
    sel = (jnp.where(iota == idxs[0], r0 * inv_norm, zero)
           + jnp.where(iota == idxs[1], r1 * inv_norm, zero)
           + jnp.where(iota == idxs[2], r2 * inv_norm, zero))

    gathered = jax.lax.dot(sel, feats_ref[0],
                           preferred_element_type=jnp.float32)
    interp = jax.lax.dot(gathered, w_ref[...],
                         preferred_element_type=jnp.float32)

    mu = jnp.mean(interp, axis=1, keepdims=True)
    xc = interp - mu
    var = jnp.mean(xc * xc, axis=1, keepdims=True)
    xn = xc / jnp.sqrt(var + 1e-6) * g_ref[0] + b_ref[0]

    inv_sqrt2 = jnp.float32(0.7071067811865476)
    act = 0.5 * xn * (1.0 + jax.lax.erf(xn * inv_sqrt2))

    out_ref[0] = act.T


def kernel(feats, coords, xyz_t, W, ln_gamma, ln_beta):
    B, M, INP = feats.shape
    _, N, _ = xyz_t.shape
    OUT = W.shape[1]
    NT = 256
    assert N % NT == 0

    coords_t = jnp.transpose(coords, (0, 2, 1))
    gamma2 = ln_gamma.reshape(1, OUT)
    beta2 = ln_beta.reshape(1, OUT)

    grid = (B, N // NT)
    out = pl.pallas_call(
        functools.partial(_ftu_kernel, nt=NT, m=M),
        grid=grid,
        in_specs=[
            pl.BlockSpec((1, 3, M), lambda b, n: (b, 0, 0)),
            pl.BlockSpec((1, NT, 3), lambda b, n: (b, n, 0)),
            pl.BlockSpec((1, M, INP), lambda b, n: (b, 0, 0)),
            pl.BlockSpec((INP, OUT), lambda b, n: (0, 0)),
            pl.BlockSpec((1, OUT), lambda b, n: (0, 0)),
            pl.BlockSpec((1, OUT), lambda b, n: (0, 0)),
        ],
        out_specs=pl.BlockSpec((1, OUT, NT), lambda b, n: (b, 0, n)),
        out_shape=jax.ShapeDtypeStruct((B, OUT, N), jnp.float32),
        compiler_params=pltpu.CompilerParams(
            dimension_semantics=("parallel", "parallel"),
        ),
    )(coords_t, xyz_t, feats, W, gamma2, beta2)
    return out


# speedup vs baseline: 46.2494x; 1.0728x over previous
import functools

import jax
import jax.numpy as jnp
from jax.experimental import pallas as pl
from jax.experimental.pallas import tpu as pltpu

QS = 0.01


def _ftu_kernel(coords_ref, xyz_ref, feats_ref, w_ref, g_ref, b_ref, out_ref,
                *, nt, m):
    src = coords_ref[0].astype(jnp.float32) * QS
    q = xyz_ref[0]

    d2 = ((q[:, 0:1] - src[0:1, :]) ** 2
          + (q[:, 1:2] - src[1:2, :]) ** 2
          + (q[:, 2:3] - src[2:3, :]) ** 2)

    iota = jax.lax.broadcasted_iota(jnp.int32, (nt, m), 1)
    inf = jnp.float32(jnp.inf)

    d = d2
    dists = []
    idxs = []
    for _ in range(3):
        mn = jnp.min(d, axis=1, keepdims=True)
        ix = jnp.min(jnp.where(d == mn, iota, m), axis=1, keepdims=True)
        dists.append(mn)
        idxs.append(ix)
        d = jnp.where(iota == ix, inf, d)

    r0 = 1.0 / (dists[0] + 1e-8)
    r1 = 1.0 / (dists[1] + 1e-8)
    r2 = 1.0 / (dists[2] + 1e-8)
    inv_norm = 1.0 / (r0 + r1 + r2)

    zero = jnp.float32(0.0)
    sel = (jnp.where(iota == idxs[0], r0 * inv_norm, zero)
           + jnp.where(iota == idxs[1], r1 * inv_norm, zero)
           + jnp.where(iota == idxs[2], r2 * inv_norm, zero))

    gathered = jax.lax.dot(sel, feats_ref[0],
                           preferred_element_type=jnp.float32)
    interp = jax.lax.dot(gathered, w_ref[...],
                         preferred_element_type=jnp.float32)

    mu = jnp.mean(interp, axis=1, keepdims=True)
    xc = interp - mu
    var = jnp.mean(xc * xc, axis=1, keepdims=True)
    xn = xc / jnp.sqrt(var + 1e-6) * g_ref[0] + b_ref[0]

    inv_sqrt2 = jnp.float32(0.7071067811865476)
    act = 0.5 * xn * (1.0 + jax.lax.erf(xn * inv_sqrt2))

    out_ref[0] = act.T


def kernel(feats, coords, xyz_t, W, ln_gamma, ln_beta):
    B, M, INP = feats.shape
    _, N, _ = xyz_t.shape
    OUT = W.shape[1]
    NT = 512
    assert N % NT == 0

    coords_t = jnp.transpose(coords, (0, 2, 1))
    gamma2 = ln_gamma.reshape(1, OUT)
    beta2 = ln_beta.reshape(1, OUT)

    grid = (B, N // NT)
    out = pl.pallas_call(
        functools.partial(_ftu_kernel, nt=NT, m=M),
        grid=grid,
        in_specs=[
            pl.BlockSpec((1, 3, M), lambda b, n: (b, 0, 0)),
            pl.BlockSpec((1, NT, 3), lambda b, n: (b, n, 0)),
            pl.BlockSpec((1, M, INP), lambda b, n: (b, 0, 0)),
            pl.BlockSpec((INP, OUT), lambda b, n: (0, 0)),
            pl.BlockSpec((1, OUT), lambda b, n: (0, 0)),
            pl.BlockSpec((1, OUT), lambda b, n: (0, 0)),
        ],
        out_specs=pl.BlockSpec((1, OUT, NT), lambda b, n: (b, 0, n)),
        out_shape=jax.ShapeDtypeStruct((B, OUT, N), jnp.float32),
        compiler_params=pltpu.CompilerParams(
            dimension_semantics=("parallel", "parallel"),
        ),
    )(coords_t, xyz_t, feats, W, gamma2, beta2)
    return out


# speedup vs baseline: 47.1398x; 1.0193x over previous
import functools

import jax
import jax.numpy as jnp
from jax.experimental import pallas as pl
from jax.experimental.pallas import tpu as pltpu

QS = 0.01


def _ftu_kernel(coords_ref, xyz_ref, feats_ref, w_ref, g_ref, b_ref, out_ref,
                *, nt, m):
    src = coords_ref[0].astype(jnp.float32) * QS
    q = xyz_ref[0]

    d2 = ((q[:, 0:1] - src[0:1, :]) ** 2
          + (q[:, 1:2] - src[1:2, :]) ** 2
          + (q[:, 2:3] - src[2:3, :]) ** 2)

    iota = jax.lax.broadcasted_iota(jnp.int32, (nt, m), 1)
    inf = jnp.float32(jnp.inf)

    d = d2
    dists = []
    idxs = []
    for _ in range(3):
        mn = jnp.min(d, axis=1, keepdims=True)
        ix = jnp.min(jnp.where(d == mn, iota, m), axis=1, keepdims=True)
        dists.append(mn)
        idxs.append(ix)
        d = jnp.where(iota == ix, inf, d)

    r0 = 1.0 / (dists[0] + 1e-8)
    r1 = 1.0 / (dists[1] + 1e-8)
    r2 = 1.0 / (dists[2] + 1e-8)
    inv_norm = 1.0 / (r0 + r1 + r2)

    zero = jnp.float32(0.0)
    sel = (jnp.where(iota == idxs[0], r0 * inv_norm, zero)
           + jnp.where(iota == idxs[1], r1 * inv_norm, zero)
           + jnp.where(iota == idxs[2], r2 * inv_norm, zero))

    gathered = jax.lax.dot(sel, feats_ref[0],
                           preferred_element_type=jnp.float32)
    interp = jax.lax.dot(gathered, w_ref[...],
                         preferred_element_type=jnp.float32)

    mu = jnp.mean(interp, axis=1, keepdims=True)
    xc = interp - mu
    var = jnp.mean(xc * xc, axis=1, keepdims=True)
    xn = xc / jnp.sqrt(var + 1e-6) * g_ref[0] + b_ref[0]

    inv_sqrt2 = jnp.float32(0.7071067811865476)
    act = 0.5 * xn * (1.0 + jax.lax.erf(xn * inv_sqrt2))

    out_ref[0] = act.T


def kernel(feats, coords, xyz_t, W, ln_gamma, ln_beta):
    B, M, INP = feats.shape
    _, N, _ = xyz_t.shape
    OUT = W.shape[1]
    NT = 1024
    assert N % NT == 0

    coords_t = jnp.transpose(coords, (0, 2, 1))
    gamma2 = ln_gamma.reshape(1, OUT)
    beta2 = ln_beta.reshape(1, OUT)

    grid = (B, N // NT)
    out = pl.pallas_call(
        functools.partial(_ftu_kernel, nt=NT, m=M),
        grid=grid,
        in_specs=[
            pl.BlockSpec((1, 3, M), lambda b, n: (b, 0, 0)),
            pl.BlockSpec((1, NT, 3), lambda b, n: (b, n, 0)),
            pl.BlockSpec((1, M, INP), lambda b, n: (b, 0, 0)),
            pl.BlockSpec((INP, OUT), lambda b, n: (0, 0)),
            pl.BlockSpec((1, OUT), lambda b, n: (0, 0)),
            pl.BlockSpec((1, OUT), lambda b, n: (0, 0)),
        ],
        out_specs=pl.BlockSpec((1, OUT, NT), lambda b, n: (b, 0, n)),
        out_shape=jax.ShapeDtypeStruct((B, OUT, N), jnp.float32),
        compiler_params=pltpu.CompilerParams(
            dimension_semantics=("parallel", "parallel"),
        ),
    )(coords_t, xyz_t, feats, W, gamma2, beta2)
    return out


# speedup vs baseline: 52.0172x; 1.1035x over previous
import functools

import jax
import jax.numpy as jnp
from jax.experimental import pallas as pl
from jax.experimental.pallas import tpu as pltpu

QS = 0.01


def _ftu_kernel(coords_ref, xyz_ref, feats_ref, w_ref, g_ref, b_ref, out_ref,
                *, nt, m):
    src = coords_ref[0].astype(jnp.float32) * QS
    q = xyz_ref[0]

    d2 = ((q[:, 0:1] - src[0:1, :]) ** 2
          + (q[:, 1:2] - src[1:2, :]) ** 2
          + (q[:, 2:3] - src[2:3, :]) ** 2)

    iota = jax.lax.broadcasted_iota(jnp.int32, (nt, m), 1).astype(jnp.float32)
    inf = jnp.float32(jnp.inf)
    mf = jnp.float32(m)

    d = d2
    dists = []
    idxs = []
    for _ in range(3):
        mn = jnp.min(d, axis=1, keepdims=True)
        ix = jnp.min(jnp.where(d == mn, iota, mf), axis=1, keepdims=True)
        dists.append(mn)
        idxs.append(ix)
        d = jnp.where(iota == ix, inf, d)

    r0 = 1.0 / (dists[0] + 1e-8)
    r1 = 1.0 / (dists[1] + 1e-8)
    r2 = 1.0 / (dists[2] + 1e-8)
    inv_norm = 1.0 / (r0 + r1 + r2)

    zero = jnp.float32(0.0)
    sel = (jnp.where(iota == idxs[0], r0 * inv_norm, zero)
           + jnp.where(iota == idxs[1], r1 * inv_norm, zero)
           + jnp.where(iota == idxs[2], r2 * inv_norm, zero))

    gathered = jax.lax.dot(sel, feats_ref[0],
                           preferred_element_type=jnp.float32)
    interp = jax.lax.dot(gathered, w_ref[...],
                         preferred_element_type=jnp.float32)

    mu = jnp.mean(interp, axis=1, keepdims=True)
    xc = interp - mu
    var = jnp.mean(xc * xc, axis=1, keepdims=True)
    xn = xc / jnp.sqrt(var + 1e-6) * g_ref[0] + b_ref[0]

    inv_sqrt2 = jnp.float32(0.7071067811865476)
    act = 0.5 * xn * (1.0 + jax.lax.erf(xn * inv_sqrt2))

    out_ref[0] = act.T


def kernel(feats, coords, xyz_t, W, ln_gamma, ln_beta):
    B, M, INP = feats.shape
    _, N, _ = xyz_t.shape
    OUT = W.shape[1]
    NT = 1024
    assert N % NT == 0

    coords_t = jnp.transpose(coords, (0, 2, 1))
    gamma2 = ln_gamma.reshape(1, OUT)
    beta2 = ln_beta.reshape(1, OUT)

    grid = (B, N // NT)
    out = pl.pallas_call(
        functools.partial(_ftu_kernel, nt=NT, m=M),
        grid=grid,
        in_specs=[
            pl.BlockSpec((1, 3, M), lambda b, n: (b, 0, 0)),
            pl.BlockSpec((1, NT, 3), lambda b, n: (b, n, 0)),
            pl.BlockSpec((1, M, INP), lambda b, n: (b, 0, 0)),
            pl.BlockSpec((INP, OUT), lambda b, n: (0, 0)),
            pl.BlockSpec((1, OUT), lambda b, n: (0, 0)),
            pl.BlockSpec((1, OUT), lambda b, n: (0, 0)),
        ],
        out_specs=pl.BlockSpec((1, OUT, NT), lambda b, n: (b, 0, n)),
        out_shape=jax.ShapeDtypeStruct((B, OUT, N), jnp.float32),
        compiler_params=pltpu.CompilerParams(
            dimension_semantics=("parallel", "parallel"),
        ),
    )(coords_t, xyz_t, feats, W, gamma2, beta2)
    return out


# speedup vs baseline: 55.1220x; 1.0597x over previous
import functools

import jax
import jax.numpy as jnp
from jax.experimental import pallas as pl
from jax.experimental.pallas import tpu as pltpu

QS = 0.01


def _ftu_kernel(coords_ref, xyz_ref, feats_ref, w_ref, g_ref, b_ref, out_ref,
                *, nt, m):
    src = coords_ref[0].astype(jnp.float32) * QS
    q = xyz_ref[0]

    d2 = ((q[:, 0:1] - src[0:1, :]) ** 2
          + (q[:, 1:2] - src[1:2, :]) ** 2
          + (q[:, 2:3] - src[2:3, :]) ** 2)

    iota = jax.lax.broadcasted_iota(jnp.int32, (nt, m), 1).astype(jnp.float32)
    inf = jnp.float32(jnp.inf)
    mf = jnp.float32(m)

    d = d2
    sel = jnp.zeros((nt, m), jnp.float32)
    rsum = jnp.zeros((nt, 1), jnp.float32)
    for k in range(3):
        mn = jnp.min(d, axis=1, keepdims=True)
        ix = jnp.min(jnp.where(d == mn, iota, mf), axis=1, keepdims=True)
        hot = iota == ix
        r = 1.0 / (mn + 1e-8)
        rsum = rsum + r
        sel = jnp.where(hot, r, sel)
        if k < 2:
            d = jnp.where(hot, inf, d)
    inv_norm = 1.0 / rsum

    gathered = jax.lax.dot(sel, feats_ref[0],
                           preferred_element_type=jnp.float32)
    interp = jax.lax.dot(gathered, w_ref[...],
                         preferred_element_type=jnp.float32) * inv_norm

    mu = jnp.mean(interp, axis=1, keepdims=True)
    xc = interp - mu
    var = jnp.mean(xc * xc, axis=1, keepdims=True)
    xn = xc / jnp.sqrt(var + 1e-6) * g_ref[0] + b_ref[0]

    inv_sqrt2 = jnp.float32(0.7071067811865476)
    act = 0.5 * xn * (1.0 + jax.lax.erf(xn * inv_sqrt2))

    out_ref[0] = act.T


def kernel(feats, coords, xyz_t, W, ln_gamma, ln_beta):
    B, M, INP = feats.shape
    _, N, _ = xyz_t.shape
    OUT = W.shape[1]
    NT = 1024
    assert N % NT == 0

    coords_t = jnp.transpose(coords, (0, 2, 1))
    gamma2 = ln_gamma.reshape(1, OUT)
    beta2 = ln_beta.reshape(1, OUT)

    grid = (B, N // NT)
    out = pl.pallas_call(
        functools.partial(_ftu_kernel, nt=NT, m=M),
        grid=grid,
        in_specs=[
            pl.BlockSpec((1, 3, M), lambda b, n: (b, 0, 0)),
            pl.BlockSpec((1, NT, 3), lambda b, n: (b, n, 0)),
            pl.BlockSpec((1, M, INP), lambda b, n: (b, 0, 0)),
            pl.BlockSpec((INP, OUT), lambda b, n: (0, 0)),
            pl.BlockSpec((1, OUT), lambda b, n: (0, 0)),
            pl.BlockSpec((1, OUT), lambda b, n: (0, 0)),
        ],
        out_specs=pl.BlockSpec((1, OUT, NT), lambda b, n: (b, 0, n)),
        out_shape=jax.ShapeDtypeStruct((B, OUT, N), jnp.float32),
        compiler_params=pltpu.CompilerParams(
            dimension_semantics=("parallel", "parallel"),
        ),
    )(coords_t, xyz_t, feats, W, gamma2, beta2)
    return out
